# Initial kernel scaffold; baseline (speedup 1.0000x reference)
import jax
import jax.numpy as jnp
from jax.experimental import pallas as pl


def kernel(x, index, v):
    raise NotImplementedError("write your pallas kernel here")


# speedup vs baseline: 21.6678x; 21.6678x over previous
import functools
import numpy as np
import jax
import jax.numpy as jnp
from jax import lax
from jax.experimental import pallas as pl
from jax.experimental.pallas import tpu as pltpu
from jax.experimental.pallas import tpu_sc as plsc

N = 50000
NT = 32
S = 32
RADIUS = 1.1
B = 64
NBK = S + 2
ACC_W = B * NBK
CH = 2000
NCH = N // CH
LANES = 16
VROW = 3 * LANES

_D = np.float32(2.0 * RADIUS / (S - 1))
_INV_D = np.float32(1.0 / _D)
_ROD = np.float32(RADIUS / _D)


def _sc_body(x0_hbm, x1_hbm, x2_hbm, idx_hbm, vcoef_hbm, cnt_hbm, sig_hbm,
             x0b, x1b, x2b, ibuf, vbuf, acc_cnt, acc_sig, red_cnt, red_sig):
    nc = 2
    wid = lax.axis_index("s") * nc + lax.axis_index("c")
    t = wid

    voff = pl.ds(pl.multiple_of(t * VROW, VROW), VROW)
    pltpu.sync_copy(vcoef_hbm.at[voff], vbuf)
    v0 = vbuf[pl.ds(0, LANES)]
    v1 = vbuf[pl.ds(LANES, LANES)]
    v2 = vbuf[pl.ds(2 * LANES, LANES)]

    zz = jnp.zeros((LANES,), jnp.float32)

    def zero_body(c, carry):
        sl = pl.ds(pl.multiple_of(c * LANES, LANES), LANES)
        for r in range(LANES):
            acc_cnt[r, sl] = zz
            acc_sig[r, sl] = zz
        return carry

    lax.fori_loop(0, ACC_W // LANES, zero_body, 0)

    lane = lax.iota(jnp.int32, LANES)
    ones = jnp.ones((LANES,), jnp.float32)

    def chunk_body(cidx, carry):
        off = pl.ds(pl.multiple_of(cidx * CH, CH), CH)
        pltpu.sync_copy(x0_hbm.at[off], x0b)
        pltpu.sync_copy(x1_hbm.at[off], x1b)
        pltpu.sync_copy(x2_hbm.at[off], x2b)
        pltpu.sync_copy(idx_hbm.at[off], ibuf)

        def vec_body(i, c2):
            sl = pl.ds(pl.multiple_of(i * LANES, LANES), LANES)
            xa = x0b[sl]
            xb = x1b[sl]
            xc = x2b[sl]
            seg = ibuf[sl]
            h = xa * v0 + xb * v1 + xc * v2
            g = h * _INV_D + _ROD
            j = (g + np.float32(128.5)).astype(jnp.int32) - 128
            jf = j.astype(jnp.float32)
            arg = np.float32(500.0) * (jf * _D - np.float32(RADIUS) - h)
            sg = np.float32(1.0) / (np.float32(1.0) + jnp.exp(-arg))
            jc = jnp.minimum(jnp.maximum(j, -1), S)
            addr = seg * NBK + (jc + 1)
            plsc.addupdate_scatter(acc_cnt, [lane, addr], ones)
            plsc.addupdate_scatter(acc_sig, [lane, addr], sg)
            return c2

        lax.fori_loop(0, CH // LANES, vec_body, 0)
        return carry

    lax.fori_loop(0, NCH, chunk_body, 0)

    def red_body(c, carry):
        sl = pl.ds(pl.multiple_of(c * LANES, LANES), LANES)
        sc = acc_cnt[0, sl]
        ss = acc_sig[0, sl]
        for r in range(1, LANES):
            sc = sc + acc_cnt[r, sl]
            ss = ss + acc_sig[r, sl]
        red_cnt[sl] = sc
        red_sig[sl] = ss
        return carry

    lax.fori_loop(0, ACC_W // LANES, red_body, 0)

    ooff = pl.ds(pl.multiple_of(t * ACC_W, ACC_W), ACC_W)
    pltpu.sync_copy(red_cnt, cnt_hbm.at[ooff])
    pltpu.sync_copy(red_sig, sig_hbm.at[ooff])


_sc_hist = functools.partial(
    pl.kernel,
    out_type=(
        jax.ShapeDtypeStruct((NT * ACC_W,), jnp.float32),
        jax.ShapeDtypeStruct((NT * ACC_W,), jnp.float32),
    ),
    mesh=plsc.VectorSubcoreMesh(
        core_axis_name="c", subcore_axis_name="s", num_cores=2, num_subcores=16),
    compiler_params=pltpu.CompilerParams(needs_layout_passes=False),
    scratch_types=(
        pltpu.VMEM((CH,), jnp.float32),
        pltpu.VMEM((CH,), jnp.float32),
        pltpu.VMEM((CH,), jnp.float32),
        pltpu.VMEM((CH,), jnp.int32),
        pltpu.VMEM((VROW,), jnp.float32),
        pltpu.VMEM((LANES, ACC_W), jnp.float32),
        pltpu.VMEM((LANES, ACC_W), jnp.float32),
        pltpu.VMEM((ACC_W,), jnp.float32),
        pltpu.VMEM((ACC_W,), jnp.float32),
    ),
)(_sc_body)


def _tc_finalize_body(d_ref, o_ref):
    row = lax.broadcasted_iota(jnp.int32, (S, 2 * NBK), 0)
    col = lax.broadcasted_iota(jnp.int32, (S, 2 * NBK), 1)
    cond = ((col <= row) & (col < NBK)) | (col == row + NBK + 1)
    g = jnp.where(cond, np.float32(1.0), np.float32(0.0))
    o_ref[...] = lax.dot_general(
        g, d_ref[...], (((1,), (1,)), ((), ())),
        preferred_element_type=jnp.float32)


def _tc_finalize(dmat):
    return pl.pallas_call(
        _tc_finalize_body,
        out_shape=jax.ShapeDtypeStruct((S, NT * B), jnp.float32),
    )(dmat)


def kernel(x, index, v):
    xt = jnp.transpose(x)
    x0 = jnp.reshape(xt[0], (N,))
    x1 = jnp.reshape(xt[1], (N,))
    x2 = jnp.reshape(xt[2], (N,))
    idx32 = index.astype(jnp.int32)
    vcoef = jnp.reshape(
        jnp.broadcast_to(jnp.transpose(v)[:, :, None], (NT, 3, LANES)),
        (NT * VROW,)).astype(jnp.float32)

    cnt, sig = _sc_hist(x0, x1, x2, idx32, vcoef)

    dmat = jnp.concatenate(
        [cnt.reshape(NT, B, NBK), sig.reshape(NT, B, NBK)], axis=-1
    ).reshape(NT * B, 2 * NBK)
    r = _tc_finalize(dmat)
    return jnp.transpose(r.reshape(S, NT, B), (2, 0, 1))


# speedup vs baseline: 54.6868x; 2.5239x over previous
import functools
import numpy as np
import jax
import jax.numpy as jnp
from jax import lax
from jax.experimental import pallas as pl
from jax.experimental.pallas import tpu as pltpu
from jax.experimental.pallas import tpu_sc as plsc

N = 50000
NT = 32
S = 32
RADIUS = 1.1
B = 64
NBK = S + 2
ACC_W = B * NBK
CH = 2000
NCH = N // CH
LANES = 16
VROW = 3 * LANES

_D = np.float32(2.0 * RADIUS / (S - 1))
_INV_D = np.float32(1.0 / _D)
_ROD = np.float32(RADIUS / _D)
_K2 = np.float32(500.0) * _D


def _sc_body(x0_hbm, x1_hbm, x2_hbm, idx_hbm, vcoef_hbm, cnt_hbm, sig_hbm,
             x0b, x1b, x2b, ibuf, vbuf, acc_cnt, acc_sig, red_cnt, red_sig,
             sem):
    nc = 2
    wid = lax.axis_index("s") * nc + lax.axis_index("c")
    t = wid

    voff = pl.ds(pl.multiple_of(t * VROW, VROW), VROW)
    pltpu.sync_copy(vcoef_hbm.at[voff], vbuf)
    v0 = vbuf[pl.ds(0, LANES)]
    v1 = vbuf[pl.ds(LANES, LANES)]
    v2 = vbuf[pl.ds(2 * LANES, LANES)]

    zz = jnp.zeros((LANES,), jnp.float32)

    @plsc.parallel_loop(0, ACC_W // LANES, unroll=4)
    def _zero(c):
        sl = pl.ds(pl.multiple_of(c * LANES, LANES), LANES)
        for r in range(LANES):
            acc_cnt[r, sl] = zz
            acc_sig[r, sl] = zz

    lane = lax.iota(jnp.int32, LANES)
    ones = jnp.ones((LANES,), jnp.float32)

    def fire(c):
        src = pl.ds(pl.multiple_of(c * CH, CH), CH)
        dst = pl.ds(pl.multiple_of((c % 2) * CH, CH), CH)
        pltpu.make_async_copy(x0_hbm.at[src], x0b.at[dst], sem).start()
        pltpu.make_async_copy(x1_hbm.at[src], x1b.at[dst], sem).start()
        pltpu.make_async_copy(x2_hbm.at[src], x2b.at[dst], sem).start()
        pltpu.make_async_copy(idx_hbm.at[src], ibuf.at[dst], sem).start()

    def drain(c):
        dst = pl.ds(pl.multiple_of((c % 2) * CH, CH), CH)
        pltpu.make_async_copy(x0_hbm.at[pl.ds(0, CH)], x0b.at[dst], sem).wait()
        pltpu.make_async_copy(x1_hbm.at[pl.ds(0, CH)], x1b.at[dst], sem).wait()
        pltpu.make_async_copy(x2_hbm.at[pl.ds(0, CH)], x2b.at[dst], sem).wait()
        pltpu.make_async_copy(idx_hbm.at[pl.ds(0, CH)], ibuf.at[dst], sem).wait()

    fire(0)

    def chunk_body(cidx, carry):
        drain(cidx)

        @pl.when(cidx + 1 < NCH)
        def _():
            fire(cidx + 1)

        pbase = (cidx % 2) * CH

        @plsc.parallel_loop(0, CH // LANES, unroll=8)
        def _vec(i):
            sl = pl.ds(pl.multiple_of(pbase + i * LANES, LANES), LANES)
            xa = x0b[sl]
            xb = x1b[sl]
            xc = x2b[sl]
            seg = ibuf[sl]
            h = xa * v0 + xb * v1 + xc * v2
            g = h * _INV_D + _ROD
            ji = (g + np.float32(128.5)).astype(jnp.int32)
            jf = ji.astype(jnp.float32) - np.float32(128.0)
            e = jnp.exp(_K2 * (g - jf))
            sg = np.float32(1.0) / (np.float32(1.0) + e)
            jc1 = jnp.minimum(jnp.maximum(ji - 127, 0), NBK - 1)
            addr = seg * NBK + jc1
            plsc.addupdate_scatter(acc_cnt, [lane, addr], ones)
            plsc.addupdate_scatter(acc_sig, [lane, addr], sg)

        return carry

    lax.fori_loop(0, NCH, chunk_body, 0)

    @plsc.parallel_loop(0, ACC_W // LANES, unroll=4)
    def _red(c):
        sl = pl.ds(pl.multiple_of(c * LANES, LANES), LANES)
        sc = acc_cnt[0, sl]
        ss = acc_sig[0, sl]
        for r in range(1, LANES):
            sc = sc + acc_cnt[r, sl]
            ss = ss + acc_sig[r, sl]
        red_cnt[sl] = sc
        red_sig[sl] = ss

    ooff = pl.ds(pl.multiple_of(t * ACC_W, ACC_W), ACC_W)
    pltpu.sync_copy(red_cnt, cnt_hbm.at[ooff])
    pltpu.sync_copy(red_sig, sig_hbm.at[ooff])


_sc_hist = functools.partial(
    pl.kernel,
    out_type=(
        jax.ShapeDtypeStruct((NT * ACC_W,), jnp.float32),
        jax.ShapeDtypeStruct((NT * ACC_W,), jnp.float32),
    ),
    mesh=plsc.VectorSubcoreMesh(
        core_axis_name="c", subcore_axis_name="s", num_cores=2, num_subcores=16),
    compiler_params=pltpu.CompilerParams(needs_layout_passes=False),
    scratch_types=(
        pltpu.VMEM((2 * CH,), jnp.float32),
        pltpu.VMEM((2 * CH,), jnp.float32),
        pltpu.VMEM((2 * CH,), jnp.float32),
        pltpu.VMEM((2 * CH,), jnp.int32),
        pltpu.VMEM((VROW,), jnp.float32),
        pltpu.VMEM((LANES, ACC_W), jnp.float32),
        pltpu.VMEM((LANES, ACC_W), jnp.float32),
        pltpu.VMEM((ACC_W,), jnp.float32),
        pltpu.VMEM((ACC_W,), jnp.float32),
        pltpu.SemaphoreType.DMA,
    ),
)(_sc_body)


def _tc_finalize_body(d_ref, o_ref):
    row = lax.broadcasted_iota(jnp.int32, (S, 2 * NBK), 0)
    col = lax.broadcasted_iota(jnp.int32, (S, 2 * NBK), 1)
    cond = ((col <= row) & (col < NBK)) | (col == row + NBK + 1)
    g = jnp.where(cond, np.float32(1.0), np.float32(0.0))
    o_ref[...] = lax.dot_general(
        g, d_ref[...], (((1,), (1,)), ((), ())),
        preferred_element_type=jnp.float32)


def _tc_finalize(dmat):
    return pl.pallas_call(
        _tc_finalize_body,
        out_shape=jax.ShapeDtypeStruct((S, NT * B), jnp.float32),
    )(dmat)


def kernel(x, index, v):
    xt = jnp.transpose(x)
    x0 = jnp.reshape(xt[0], (N,))
    x1 = jnp.reshape(xt[1], (N,))
    x2 = jnp.reshape(xt[2], (N,))
    idx32 = index.astype(jnp.int32)
    vcoef = jnp.reshape(
        jnp.broadcast_to(jnp.transpose(v)[:, :, None], (NT, 3, LANES)),
        (NT * VROW,)).astype(jnp.float32)

    cnt, sig = _sc_hist(x0, x1, x2, idx32, vcoef)

    dmat = jnp.concatenate(
        [cnt.reshape(NT, B, NBK), sig.reshape(NT, B, NBK)], axis=-1
    ).reshape(NT * B, 2 * NBK)
    r = _tc_finalize(dmat)
    return jnp.transpose(r.reshape(S, NT, B), (2, 0, 1))


# speedup vs baseline: 56.1800x; 1.0273x over previous
import functools
import numpy as np
import jax
import jax.numpy as jnp
from jax import lax
from jax.experimental import pallas as pl
from jax.experimental.pallas import tpu as pltpu
from jax.experimental.pallas import tpu_sc as plsc

N = 50000
NT = 32
S = 32
RADIUS = 1.1
B = 64
NBK = S + 2
ACC_W = B * NBK
ACC_P = ACC_W + 1
CH = 2000
NCH = N // CH
LANES = 16
VROW = 3 * LANES

_D = np.float32(2.0 * RADIUS / (S - 1))
_INV_D = np.float32(1.0 / _D)
_ROD = np.float32(RADIUS / _D)
_K2 = np.float32(500.0) * _D


def _sc_body(x0_hbm, x1_hbm, x2_hbm, idx_hbm, vcoef_hbm, cnt_hbm, sig_hbm,
             x0b, x1b, x2b, ibuf, vbuf, acc_cnt, acc_sig, red_cnt, red_sig,
             sem):
    nc = 2
    wid = lax.axis_index("s") * nc + lax.axis_index("c")
    t = wid

    def fire(c):
        src = pl.ds(pl.multiple_of(c * CH, CH), CH)
        dst = pl.ds(pl.multiple_of((c % 2) * CH, CH), CH)
        pltpu.make_async_copy(x0_hbm.at[src], x0b.at[dst], sem).start()
        pltpu.make_async_copy(x1_hbm.at[src], x1b.at[dst], sem).start()
        pltpu.make_async_copy(x2_hbm.at[src], x2b.at[dst], sem).start()
        pltpu.make_async_copy(idx_hbm.at[src], ibuf.at[dst], sem).start()

    def drain(c):
        dst = pl.ds(pl.multiple_of((c % 2) * CH, CH), CH)
        pltpu.make_async_copy(x0_hbm.at[pl.ds(0, CH)], x0b.at[dst], sem).wait()
        pltpu.make_async_copy(x1_hbm.at[pl.ds(0, CH)], x1b.at[dst], sem).wait()
        pltpu.make_async_copy(x2_hbm.at[pl.ds(0, CH)], x2b.at[dst], sem).wait()
        pltpu.make_async_copy(idx_hbm.at[pl.ds(0, CH)], ibuf.at[dst], sem).wait()

    fire(0)

    voff = pl.ds(pl.multiple_of(t * VROW, VROW), VROW)
    pltpu.sync_copy(vcoef_hbm.at[voff], vbuf)
    v0 = vbuf[pl.ds(0, LANES)]
    v1 = vbuf[pl.ds(LANES, LANES)]
    v2 = vbuf[pl.ds(2 * LANES, LANES)]

    zz = jnp.zeros((LANES,), jnp.float32)

    @plsc.parallel_loop(0, ACC_W // LANES, unroll=4)
    def _zero(c):
        sl = pl.ds(pl.multiple_of(c * LANES, LANES), LANES)
        for r in range(LANES):
            acc_cnt[r, sl] = zz
            acc_sig[r, sl] = zz

    lane = lax.iota(jnp.int32, LANES)
    ones = jnp.ones((LANES,), jnp.float32)

    def chunk_body(cidx, carry):
        drain(cidx)

        @pl.when(cidx + 1 < NCH)
        def _():
            fire(cidx + 1)

        pbase = (cidx % 2) * CH

        @plsc.parallel_loop(0, CH // LANES, unroll=16)
        def _vec(i):
            sl = pl.ds(pl.multiple_of(pbase + i * LANES, LANES), LANES)
            xa = x0b[sl]
            xb = x1b[sl]
            xc = x2b[sl]
            seg = ibuf[sl]
            h = xa * v0 + xb * v1 + xc * v2
            g = h * _INV_D + _ROD
            ji = (g + np.float32(128.5)).astype(jnp.int32)
            jf = ji.astype(jnp.float32) - np.float32(128.0)
            e = jnp.exp(_K2 * (g - jf))
            sg = np.float32(1.0) / (np.float32(1.0) + e)
            jc1 = jnp.minimum(jnp.maximum(ji - 127, 0), NBK - 1)
            addr = seg * NBK + jc1
            plsc.addupdate_scatter(acc_cnt, [lane, addr], ones)
            plsc.addupdate_scatter(acc_sig, [lane, addr], sg)

        return carry

    lax.fori_loop(0, NCH, chunk_body, 0)

    @plsc.parallel_loop(0, ACC_W // LANES, unroll=4)
    def _red(c):
        sl = pl.ds(pl.multiple_of(c * LANES, LANES), LANES)
        sc = acc_cnt[0, sl]
        ss = acc_sig[0, sl]
        for r in range(1, LANES):
            sc = sc + acc_cnt[r, sl]
            ss = ss + acc_sig[r, sl]
        red_cnt[sl] = sc
        red_sig[sl] = ss

    ooff = pl.ds(pl.multiple_of(t * ACC_W, ACC_W), ACC_W)
    pltpu.sync_copy(red_cnt, cnt_hbm.at[ooff])
    pltpu.sync_copy(red_sig, sig_hbm.at[ooff])


_sc_hist = functools.partial(
    pl.kernel,
    out_type=(
        jax.ShapeDtypeStruct((NT * ACC_W,), jnp.float32),
        jax.ShapeDtypeStruct((NT * ACC_W,), jnp.float32),
    ),
    mesh=plsc.VectorSubcoreMesh(
        core_axis_name="c", subcore_axis_name="s", num_cores=2, num_subcores=16),
    compiler_params=pltpu.CompilerParams(needs_layout_passes=False),
    scratch_types=(
        pltpu.VMEM((2 * CH,), jnp.float32),
        pltpu.VMEM((2 * CH,), jnp.float32),
        pltpu.VMEM((2 * CH,), jnp.float32),
        pltpu.VMEM((2 * CH,), jnp.int32),
        pltpu.VMEM((VROW,), jnp.float32),
        pltpu.VMEM((LANES, ACC_P), jnp.float32),
        pltpu.VMEM((LANES, ACC_P), jnp.float32),
        pltpu.VMEM((ACC_W,), jnp.float32),
        pltpu.VMEM((ACC_W,), jnp.float32),
        pltpu.SemaphoreType.DMA,
    ),
)(_sc_body)


def _tc_finalize_body(d_ref, o_ref):
    row = lax.broadcasted_iota(jnp.int32, (S, 2 * NBK), 0)
    col = lax.broadcasted_iota(jnp.int32, (S, 2 * NBK), 1)
    cond = ((col <= row) & (col < NBK)) | (col == row + NBK + 1)
    g = jnp.where(cond, np.float32(1.0), np.float32(0.0))
    o_ref[...] = lax.dot_general(
        g, d_ref[...], (((1,), (1,)), ((), ())),
        preferred_element_type=jnp.float32)


def _tc_finalize(dmat):
    return pl.pallas_call(
        _tc_finalize_body,
        out_shape=jax.ShapeDtypeStruct((S, NT * B), jnp.float32),
    )(dmat)


def kernel(x, index, v):
    xt = jnp.transpose(x)
    x0 = jnp.reshape(xt[0], (N,))
    x1 = jnp.reshape(xt[1], (N,))
    x2 = jnp.reshape(xt[2], (N,))
    idx32 = index.astype(jnp.int32)
    vcoef = jnp.reshape(
        jnp.broadcast_to(jnp.transpose(v)[:, :, None], (NT, 3, LANES)),
        (NT * VROW,)).astype(jnp.float32)

    cnt, sig = _sc_hist(x0, x1, x2, idx32, vcoef)

    dmat = jnp.concatenate(
        [cnt.reshape(NT, B, NBK), sig.reshape(NT, B, NBK)], axis=-1
    ).reshape(NT * B, 2 * NBK)
    r = _tc_finalize(dmat)
    return jnp.transpose(r.reshape(S, NT, B), (2, 0, 1))
